# Initial kernel scaffold; baseline (speedup 1.0000x reference)
import jax
import jax.numpy as jnp
from jax.experimental import pallas as pl


def kernel(tokens_1gram, tokens_2gram, tokens_3gram, emb1, emb2, emb3, W1, b1, W2, b2):
    raise NotImplementedError("write your pallas kernel here")


# speedup vs baseline: 4.9733x; 4.9733x over previous
import functools

import jax
import jax.numpy as jnp
from jax import lax
from jax.experimental import pallas as pl
from jax.experimental.pallas import tpu as pltpu
from jax.experimental.pallas import tpu_sc as plsc

B = 4096
S = 200
D = 64
L = 16
CHUNK = 40
CPR = S // CHUNK
NW = 32
BPW = B // NW
TASKS = BPW * CPR
DV = D // L


def _pool_body(tok1, tok2, tok3, emb1, emb2, emb3, out, idx_v, rows0, rows1, stage, sem):
    cid = lax.axis_index("c")
    sid = lax.axis_index("s")
    wid = sid * 2 + cid

    def zbody(i, carry):
        z = jnp.zeros((L,), jnp.float32)
        for j in range(3 * D // L):
            stage[i, pl.ds(L * j, L)] = z
        return carry

    lax.fori_loop(0, BPW, zbody, 0)

    for t, (tok, emb) in enumerate(((tok1, emb1), (tok2, emb2), (tok3, emb3))):
        pltpu.sync_copy(tok.at[pl.ds(wid * TASKS, TASKS)], idx_v)

        def fire(k, rbuf, emb=emb):
            pltpu.make_async_copy(emb.at[idx_v.at[k]], rbuf, sem).start()

        def drain(k, rbuf, emb=emb):
            pltpu.make_async_copy(emb.at[idx_v.at[k]], rbuf, sem).wait()

        def accum(k, rbuf, t=t):
            acc = [jnp.zeros((L,), jnp.float32) for _ in range(2 * DV)]
            for s in range(CHUNK):
                bank = (s % 2) * DV
                for j in range(DV):
                    acc[bank + j] = acc[bank + j] + rbuf[s, pl.ds(L * j, L)]
            b_loc = k // CPR
            for j in range(DV):
                plsc.addupdate(
                    stage.at[b_loc, pl.ds(t * D + L * j, L)], acc[j] + acc[DV + j]
                )

        fire(0, rows0)

        def lbody(kk, carry):
            k0 = 2 * kk
            fire(k0 + 1, rows1)
            drain(k0, rows0)
            accum(k0, rows0)

            @pl.when(kk < TASKS // 2 - 1)
            def _():
                fire(k0 + 2, rows0)

            drain(k0 + 1, rows1)
            accum(k0 + 1, rows1)
            return carry

        lax.fori_loop(0, TASKS // 2, lbody, 0)

    pltpu.sync_copy(stage, out.at[pl.ds(wid * BPW, BPW)])


_pool = functools.partial(
    pl.kernel,
    out_type=jax.ShapeDtypeStruct((B, 3 * D), jnp.float32),
    mesh=plsc.VectorSubcoreMesh(core_axis_name="c", subcore_axis_name="s"),
    scratch_types=[
        pltpu.VMEM((TASKS, CHUNK), jnp.int32),
        pltpu.VMEM((CHUNK, D), jnp.float32),
        pltpu.VMEM((CHUNK, D), jnp.float32),
        pltpu.VMEM((BPW, 3 * D), jnp.float32),
        pltpu.SemaphoreType.DMA,
    ],
    compiler_params=pltpu.CompilerParams(use_tc_tiling_on_sc=False),
)(_pool_body)


def _mlp_body(x_ref, w1_ref, b1_ref, w2_ref, b2_ref, o_ref):
    x = x_ref[...]
    h = lax.dot_general(
        x, w1_ref[...], (((1,), (0,)), ((), ())),
        preferred_element_type=jnp.float32, precision=lax.Precision.HIGHEST,
    )
    h = jnp.maximum(h * (1.0 / S) + b1_ref[...], 0.0)
    o = lax.dot_general(
        h, w2_ref[...], (((1,), (0,)), ((), ())),
        preferred_element_type=jnp.float32, precision=lax.Precision.HIGHEST,
    )
    o_ref[...] = o + b2_ref[...]


def _mlp(pooled, W1, b1, W2, b2):
    return pl.pallas_call(
        _mlp_body,
        out_shape=jax.ShapeDtypeStruct((B, W2.shape[1]), jnp.float32),
    )(pooled, W1, b1.reshape(1, -1), W2, b2.reshape(1, -1))


def kernel(tokens_1gram, tokens_2gram, tokens_3gram, emb1, emb2, emb3, W1, b1, W2, b2):
    t1 = tokens_1gram.reshape(-1, CHUNK)
    t2 = tokens_2gram.reshape(-1, CHUNK)
    t3 = tokens_3gram.reshape(-1, CHUNK)
    pooled = _pool(t1, t2, t3, emb1, emb2, emb3)
    return _mlp(pooled, W1, b1, W2, b2)
